# Initial kernel scaffold; baseline (speedup 1.0000x reference)
import jax
import jax.numpy as jnp
from jax.experimental import pallas as pl


def kernel(input_ids, segment_ids, token_table, segment_table, pe):
    raise NotImplementedError("write your pallas kernel here")


# speedup vs baseline: 7.4123x; 7.4123x over previous
import jax
import jax.numpy as jnp
from jax import lax
from jax.experimental import pallas as pl
from jax.experimental.pallas import tpu as pltpu
from jax.experimental.pallas import tpu_sc as plsc

VOCAB = 100000
EMBED = 128
SEQ = 200
BATCH = 1024

_INFO = plsc.get_sparse_core_info()
NC = _INFO.num_cores
NS = _INFO.num_subcores
L = _INFO.num_lanes
NW = NC * NS
NJ = EMBED // L

N = BATCH * SEQ
CHUNK = 80
ROWS_PER_W = N // NW
CHUNKS_PER_W = ROWS_PER_W // CHUNK
PE_PAD = SEQ + 8


def _body(ids_hbm, segs_hbm, tok_tab_hbm, seg_tab_hbm, pe_hbm, out_hbm,
          idx_v, segid_v, tok_v, out_v, pe_v, seg_v, gsem, osem):
    wid = lax.axis_index("s") * NC + lax.axis_index("c")

    pltpu.sync_copy(pe_hbm, pe_v)
    pltpu.sync_copy(seg_tab_hbm, seg_v)

    s1 = [seg_v[1, pl.ds(L * j, L)] for j in range(NJ)]
    s2 = [seg_v[2, pl.ds(L * j, L)] for j in range(NJ)]

    def stage_and_issue(c, b):
        blk = wid * CHUNKS_PER_W + c
        pltpu.sync_copy(ids_hbm.at[pl.ds(blk, 1)], idx_v.at[b])
        pltpu.sync_copy(segs_hbm.at[pl.ds(blk * CHUNK, CHUNK)],
                        segid_v.at[pl.ds(b * CHUNK, CHUNK)])
        pltpu.async_copy(tok_tab_hbm.at[idx_v.at[b].at[0]], tok_v.at[b],
                         gsem.at[b])

    def compute(c, b):
        base = (wid * CHUNKS_PER_W + c) * CHUNK
        pltpu.make_async_copy(
            tok_tab_hbm.at[idx_v.at[b].at[0]], tok_v.at[b], gsem.at[b]).wait()

        @plsc.parallel_loop(0, CHUNK, unroll=16)
        def row_body(r):
            off = lax.rem(base + r, SEQ)
            sid = segid_v[pl.ds(b * CHUNK + r, L)][0]
            m1 = sid == 1
            m2 = sid == 2
            zero = jnp.zeros((L,), jnp.float32)
            for j in range(NJ):
                t = tok_v[b, r, pl.ds(L * j, L)]
                p = pe_v[off, pl.ds(L * j, L)]
                addon = jnp.where(m1, s1[j], jnp.where(m2, s2[j], zero))
                out_v[b, r, pl.ds(L * j, L)] = t + p + addon

        pltpu.async_copy(out_v.at[b], out_hbm.at[pl.ds(base, CHUNK)], osem.at[b])

    def wait_out(c, b):
        base = (wid * CHUNKS_PER_W + c) * CHUNK
        pltpu.make_async_copy(
            out_v.at[b], out_hbm.at[pl.ds(base, CHUNK)], osem.at[b]).wait()

    stage_and_issue(0, 0)

    def pair_body(c2, _):
        c = c2 * 2
        stage_and_issue(c + 1, 1)

        @pl.when(c2 > 0)
        def _():
            wait_out(c - 2, 0)

        compute(c, 0)

        @pl.when(c + 2 < CHUNKS_PER_W)
        def _():
            stage_and_issue(c + 2, 0)

        @pl.when(c2 > 0)
        def _():
            wait_out(c - 1, 1)

        compute(c + 1, 1)
        return ()

    lax.fori_loop(0, CHUNKS_PER_W // 2, pair_body, ())
    wait_out(CHUNKS_PER_W - 2, 0)
    wait_out(CHUNKS_PER_W - 1, 1)


@jax.jit
def _run(ids2, segs2, token_table, segment_table, pe_pad):
    mesh = plsc.VectorSubcoreMesh(core_axis_name="c", subcore_axis_name="s")
    kfn = pl.kernel(
        _body,
        out_type=jax.ShapeDtypeStruct((N, EMBED), jnp.float32),
        mesh=mesh,
        scratch_types=[
            pltpu.VMEM((2, 1, CHUNK), jnp.int32),
            pltpu.VMEM((2 * CHUNK + L,), jnp.int32),
            pltpu.VMEM((2, CHUNK, EMBED), jnp.float32),
            pltpu.VMEM((2, CHUNK, EMBED), jnp.float32),
            pltpu.VMEM((PE_PAD, EMBED), jnp.float32),
            pltpu.VMEM((3, EMBED), jnp.float32),
            pltpu.SemaphoreType.DMA((2,)),
            pltpu.SemaphoreType.DMA((2,)),
        ],
    )
    return kfn(ids2, segs2, token_table, segment_table, pe_pad)


def kernel(input_ids, segment_ids, token_table, segment_table, pe):
    ids2 = input_ids.reshape(N // CHUNK, CHUNK).astype(jnp.int32)
    segs2 = segment_ids.reshape(N).astype(jnp.int32)
    pe2 = pe.reshape(SEQ, EMBED)
    pe_pad = jnp.concatenate([pe2, pe2[: PE_PAD - SEQ]], axis=0)
    out = _run(ids2, segs2, token_table, segment_table, pe_pad)
    return out.reshape(BATCH, SEQ, EMBED)
